# baseline (speedup 1.0000x reference)
import functools

import jax
import jax.numpy as jnp
from jax import lax
from jax.experimental import pallas as pl
from jax.experimental.pallas import tpu as pltpu
from jax.experimental.pallas import tpu_sc as plsc


_CH = 128
_NTILES = 32


def _npad(n):
    return ((n + 1 + 127) // 128) * 128


def _stage_mid_body(agg_ref, deg_ref, b_ref, w_ref, o_ref):
    n = o_ref.shape[0]
    agg = agg_ref[0, :n] + agg_ref[1, :n]
    deg = deg_ref[0, :n, :1] + deg_ref[1, :n, :1]
    inv = 1.0 / jnp.maximum(deg, 1.0)
    x = jnp.dot(agg * inv, w_ref[...], preferred_element_type=jnp.float32)
    o_ref[...] = jnp.maximum(x + b_ref[...], 0.0)


def _tc_stage_mid(aggp, degp, n, b, w):
    return pl.pallas_call(
        _stage_mid_body,
        out_shape=jax.ShapeDtypeStruct((n, w.shape[1]), jnp.float32),
    )(aggp, degp, b.reshape(1, w.shape[1]), w)


def _stage_out_body(n, agg_ref, deg_ref, w_ref, b_ref, nn_row_ref, nn_col_ref,
                    wout_ref, bout_ref, o_ref):
    agg = agg_ref[0, :n] + agg_ref[1, :n]
    deg = deg_ref[0, :n, :1] + deg_ref[1, :n, :1]
    inv = 1.0 / jnp.maximum(deg, 1.0)
    xw = jnp.dot(agg * inv, w_ref[...], preferred_element_type=jnp.float32)
    x = jnp.maximum(xw + b_ref[...], 0.0)

    g = nn_row_ref.shape[1]
    nn = nn_row_ref[...]
    r = lax.broadcasted_iota(jnp.int32, (g, g), 0)
    c = lax.broadcasted_iota(jnp.int32, (g, g), 1)
    lt = (r <= c).astype(jnp.float32)
    incl = jnp.dot(nn, lt, preferred_element_type=jnp.float32)
    start = incl - nn
    gidx = lax.broadcasted_iota(jnp.int32, (1, g), 1)
    end = jnp.where(gidx == g - 1, jnp.float32(1 << 30), incl)
    p = lax.broadcasted_iota(jnp.int32, (n, g), 0).astype(jnp.float32)
    onehot = ((p >= start) & (p < end)).astype(jnp.float32)
    pooled_sum = lax.dot_general(
        onehot, x, (((0,), (0,)), ((), ())),
        preferred_element_type=jnp.float32)
    pooled = pooled_sum / jnp.maximum(nn_col_ref[...], 1.0)
    o_ref[...] = jnp.dot(pooled, wout_ref[...],
                         preferred_element_type=jnp.float32) + bout_ref[...]


def _tc_stage_out(aggp, degp, n, w, b, n_node, w_out, b_out):
    g = n_node.shape[0]
    nn_f = n_node.astype(jnp.float32)
    return pl.pallas_call(
        functools.partial(_stage_out_body, n),
        out_shape=jax.ShapeDtypeStruct((g, w_out.shape[1]), jnp.float32),
    )(aggp, degp, w, b.reshape(1, w.shape[1]), nn_f.reshape(1, g),
      nn_f.reshape(g, 1), w_out, b_out.reshape(1, w_out.shape[1]))


@functools.lru_cache(maxsize=None)
def _make_sc_agg(n, d, n_chunks):
    mesh = plsc.VectorSubcoreMesh(core_axis_name="c", subcore_axis_name="s")
    ncores, nsub = mesh.num_cores, mesh.num_subcores
    npad = _npad(n)
    rpt = npad // nsub
    assert rpt % 8 == 0

    assert n_chunks % 2 == 0 and n_chunks >= 4
    pairs = n_chunks // 2

    def body(y_hbm, sidx_hbm, ridx_hbm, z_hbm, agg_hbm,
             acc, sidx0, ridx0, gbuf0, sidx1, ridx1, gbuf1, sem0, sem1):
        c = lax.axis_index("c")
        s = lax.axis_index("s")
        w = c * nsub + s
        base = s * rpt

        pltpu.sync_copy(z_hbm, acc.at[pl.ds(base, rpt)])
        plsc.subcore_barrier()

        pltpu.sync_copy(sidx_hbm.at[w, 0], sidx0)
        pltpu.sync_copy(ridx_hbm.at[w, 0], ridx0)
        pltpu.async_copy(y_hbm.at[sidx0], gbuf0, sem0)

        def pair(jj, carry):
            j1 = 2 * jj + 1
            pltpu.sync_copy(sidx_hbm.at[w, j1], sidx1)
            pltpu.sync_copy(ridx_hbm.at[w, j1], ridx1)
            pltpu.async_copy(y_hbm.at[sidx1], gbuf1, sem1)
            pltpu.make_async_copy(y_hbm.at[pl.ds(0, _CH)], gbuf0, sem0).wait()
            pltpu.sync_copy(gbuf0, acc.at[ridx0], add=True)
            jn = jnp.minimum(2 * jj + 2, n_chunks - 2)
            pltpu.sync_copy(sidx_hbm.at[w, jn], sidx0)
            pltpu.sync_copy(ridx_hbm.at[w, jn], ridx0)
            pltpu.async_copy(y_hbm.at[sidx0], gbuf0, sem0)
            pltpu.make_async_copy(y_hbm.at[pl.ds(0, _CH)], gbuf1, sem1).wait()
            pltpu.sync_copy(gbuf1, acc.at[ridx1], add=True)
            return carry

        lax.fori_loop(0, pairs, pair, 0)
        pltpu.make_async_copy(y_hbm.at[pl.ds(0, _CH)], gbuf0, sem0).wait()

        plsc.subcore_barrier()
        pltpu.sync_copy(acc.at[pl.ds(base, rpt)],
                        agg_hbm.at[c, pl.ds(base, rpt)])

    return pl.kernel(
        body,
        out_type=jax.ShapeDtypeStruct((ncores, npad, d), jnp.float32),
        mesh=mesh,
        scratch_types=(
            pltpu.VMEM_SHARED((npad, d), jnp.float32),
            pltpu.VMEM((_CH,), jnp.int32),
            pltpu.VMEM((_CH,), jnp.int32),
            pltpu.VMEM((_CH, d), jnp.float32),
            pltpu.VMEM((_CH,), jnp.int32),
            pltpu.VMEM((_CH,), jnp.int32),
            pltpu.VMEM((_CH, d), jnp.float32),
            pltpu.SemaphoreType.DMA,
            pltpu.SemaphoreType.DMA,
        )), npad, rpt


def _sc_aggregate(y, sidx, ridx):
    n, d = y.shape
    n_chunks = sidx.shape[1]
    k, npad, rpt = _make_sc_agg(n, d, n_chunks)
    z = jnp.zeros((rpt, d), jnp.float32)
    return k(y, sidx, ridx, z)


@functools.lru_cache(maxsize=None)
def _make_sc_deg(n, d, n_chunks):
    mesh = plsc.VectorSubcoreMesh(core_axis_name="c", subcore_axis_name="s")
    ncores, nsub = mesh.num_cores, mesh.num_subcores
    npad = _npad(n)
    rpt = npad // nsub
    assert rpt % 8 == 0

    def body(ridx_hbm, z_hbm, ones_hbm, deg_hbm, acc, ridx_v, ones_buf):
        c = lax.axis_index("c")
        s = lax.axis_index("s")
        w = c * nsub + s
        base = s * rpt

        pltpu.sync_copy(z_hbm, acc.at[pl.ds(base, rpt)])
        pltpu.sync_copy(ones_hbm, ones_buf)
        plsc.subcore_barrier()

        def chunk(j, carry):
            pltpu.sync_copy(ridx_hbm.at[w, j], ridx_v)
            pltpu.sync_copy(ones_buf, acc.at[ridx_v], add=True)
            return carry

        lax.fori_loop(0, n_chunks, chunk, 0)
        plsc.subcore_barrier()
        pltpu.sync_copy(acc.at[pl.ds(base, rpt)],
                        deg_hbm.at[c, pl.ds(base, rpt)])

    return pl.kernel(
        body,
        out_type=jax.ShapeDtypeStruct((ncores, npad, d), jnp.float32),
        mesh=mesh,
        scratch_types=(
            pltpu.VMEM_SHARED((npad, d), jnp.float32),
            pltpu.VMEM((_CH,), jnp.int32),
            pltpu.VMEM((_CH, d), jnp.float32),
        )), npad, rpt


def _sc_degree(ridx, n, d):
    n_chunks = ridx.shape[1]
    k, npad, rpt = _make_sc_deg(n, d, n_chunks)
    z = jnp.zeros((rpt, d), jnp.float32)
    ones_blk = jnp.ones((_CH, d), jnp.float32)
    return k(ridx, z, ones_blk)


def _edge_blocks(senders, receivers, n):
    e = senders.shape[0]
    per_tile = -(-e // (_NTILES * _CH * 2)) * 2
    e_pad = _NTILES * per_tile * _CH
    pad = e_pad - e
    sidx = jnp.concatenate(
        [senders, jnp.zeros((pad,), jnp.int32)]).reshape(_NTILES, per_tile, _CH)
    ridx = jnp.concatenate(
        [receivers, jnp.full((pad,), n, jnp.int32)]).reshape(_NTILES, per_tile, _CH)
    return sidx, ridx


def kernel(nodes, senders, receivers, n_node, W1, b1, W2, b2, W_out, b_out):
    n, d = nodes.shape
    sidx, ridx = _edge_blocks(senders, receivers, n)

    deg_full = _sc_degree(ridx, n, d)
    degp = deg_full[:, :, :16]
    agg1p = _sc_aggregate(nodes, sidx, ridx)
    x1 = _tc_stage_mid(agg1p, degp, n, b1, W1)
    agg2p = _sc_aggregate(x1, sidx, ridx)
    return _tc_stage_out(agg2p, degp, n, W2, b2, n_node, W_out, b_out)


# speedup vs baseline: 1.7548x; 1.7548x over previous
import functools

import jax
import jax.numpy as jnp
from jax import lax
from jax.experimental import pallas as pl
from jax.experimental.pallas import tpu as pltpu
from jax.experimental.pallas import tpu_sc as plsc


_CH = 128
_NTILES = 32
_NCA = 112
_NCB = 46


def _npad(n):
    return ((n + 1 + 127) // 128) * 128


def _stage_mid_body(agg_ref, deg_ref, b_ref, w_ref, o_ref):
    n = o_ref.shape[0]
    agg = agg_ref[0, :n] + agg_ref[1, :n]
    deg = deg_ref[0, :n, :1] + deg_ref[1, :n, :1]
    inv = 1.0 / jnp.maximum(deg, 1.0)
    x = jnp.dot(agg * inv, w_ref[...], preferred_element_type=jnp.float32)
    o_ref[...] = jnp.maximum(x + b_ref[...], 0.0)


def _tc_stage_mid(aggp, degp, n, b, w):
    return pl.pallas_call(
        _stage_mid_body,
        out_shape=jax.ShapeDtypeStruct((n, w.shape[1]), jnp.float32),
    )(aggp, degp, b.reshape(1, w.shape[1]), w)


def _stage_out_body(n, agg_ref, deg_ref, w_ref, b_ref, nn_row_ref, nn_col_ref,
                    wout_ref, bout_ref, o_ref):
    agg = agg_ref[0, :n] + agg_ref[1, :n]
    deg = deg_ref[0, :n, :1] + deg_ref[1, :n, :1]
    inv = 1.0 / jnp.maximum(deg, 1.0)
    xw = jnp.dot(agg * inv, w_ref[...], preferred_element_type=jnp.float32)
    x = jnp.maximum(xw + b_ref[...], 0.0)

    g = nn_row_ref.shape[1]
    nn = nn_row_ref[...]
    r = lax.broadcasted_iota(jnp.int32, (g, g), 0)
    c = lax.broadcasted_iota(jnp.int32, (g, g), 1)
    lt = (r <= c).astype(jnp.float32)
    incl = jnp.dot(nn, lt, preferred_element_type=jnp.float32)
    start = incl - nn
    gidx = lax.broadcasted_iota(jnp.int32, (1, g), 1)
    end = jnp.where(gidx == g - 1, jnp.float32(1 << 30), incl)
    p = lax.broadcasted_iota(jnp.int32, (n, g), 0).astype(jnp.float32)
    onehot = ((p >= start) & (p < end)).astype(jnp.float32)
    pooled_sum = lax.dot_general(
        onehot, x, (((0,), (0,)), ((), ())),
        preferred_element_type=jnp.float32)
    pooled = pooled_sum / jnp.maximum(nn_col_ref[...], 1.0)
    o_ref[...] = jnp.dot(pooled, wout_ref[...],
                         preferred_element_type=jnp.float32) + bout_ref[...]


def _tc_stage_out(aggp, degp, n, w, b, n_node, w_out, b_out):
    g = n_node.shape[0]
    nn_f = n_node.astype(jnp.float32)
    return pl.pallas_call(
        functools.partial(_stage_out_body, n),
        out_shape=jax.ShapeDtypeStruct((g, w_out.shape[1]), jnp.float32),
    )(aggp, degp, w, b.reshape(1, w.shape[1]), nn_f.reshape(1, g),
      nn_f.reshape(g, 1), w_out, b_out.reshape(1, w_out.shape[1]))


@functools.lru_cache(maxsize=None)
def _make_sc_agg(n, d, nca, ncb):
    mesh = plsc.VectorSubcoreMesh(core_axis_name="c", subcore_axis_name="s")
    ncores, nsub = mesh.num_cores, mesh.num_subcores
    npad = _npad(n)
    rpt = npad // nsub
    assert rpt % 8 == 0
    assert nca % 2 == 0 and ncb % 2 == 0 and nca >= 4 and ncb >= 4

    def body(y_hbm, sidx_hbm, ridx_hbm, z_hbm, agg_hbm,
             acc, sidx0, ridx0, gbuf0, sidx1, ridx1, gbuf1, sem0, sem1):
        c = lax.axis_index("c")
        s = lax.axis_index("s")
        base = s * rpt
        cb = jnp.where(c == 0, s * nca, nsub * nca + s * ncb)
        n_loc = jnp.where(c == 0, nca, ncb)
        pairs = n_loc // 2

        pltpu.sync_copy(z_hbm, acc.at[pl.ds(base, rpt)])
        plsc.subcore_barrier()

        pltpu.sync_copy(sidx_hbm.at[cb], sidx0)
        pltpu.sync_copy(ridx_hbm.at[cb], ridx0)
        pltpu.async_copy(y_hbm.at[sidx0], gbuf0, sem0)

        def pair(jj, carry):
            j1 = cb + 2 * jj + 1
            pltpu.sync_copy(sidx_hbm.at[j1], sidx1)
            pltpu.sync_copy(ridx_hbm.at[j1], ridx1)
            pltpu.async_copy(y_hbm.at[sidx1], gbuf1, sem1)
            pltpu.make_async_copy(y_hbm.at[pl.ds(0, _CH)], gbuf0, sem0).wait()
            pltpu.sync_copy(gbuf0, acc.at[ridx0], add=True)
            jn = cb + jnp.minimum(2 * jj + 2, n_loc - 2)
            pltpu.sync_copy(sidx_hbm.at[jn], sidx0)
            pltpu.sync_copy(ridx_hbm.at[jn], ridx0)
            pltpu.async_copy(y_hbm.at[sidx0], gbuf0, sem0)
            pltpu.make_async_copy(y_hbm.at[pl.ds(0, _CH)], gbuf1, sem1).wait()
            pltpu.sync_copy(gbuf1, acc.at[ridx1], add=True)
            return carry

        lax.fori_loop(0, pairs, pair, 0)
        pltpu.make_async_copy(y_hbm.at[pl.ds(0, _CH)], gbuf0, sem0).wait()

        plsc.subcore_barrier()
        pltpu.sync_copy(acc.at[pl.ds(base, rpt)],
                        agg_hbm.at[c, pl.ds(base, rpt)])

    return pl.kernel(
        body,
        out_type=jax.ShapeDtypeStruct((ncores, npad, d), jnp.float32),
        mesh=mesh,
        scratch_types=(
            pltpu.VMEM_SHARED((npad, d), jnp.float32),
            pltpu.VMEM((_CH,), jnp.int32),
            pltpu.VMEM((_CH,), jnp.int32),
            pltpu.VMEM((_CH, d), jnp.float32),
            pltpu.VMEM((_CH,), jnp.int32),
            pltpu.VMEM((_CH,), jnp.int32),
            pltpu.VMEM((_CH, d), jnp.float32),
            pltpu.SemaphoreType.DMA,
            pltpu.SemaphoreType.DMA,
        )), npad, rpt


def _sc_aggregate(y, sidx, ridx):
    n, d = y.shape
    k, npad, rpt = _make_sc_agg(n, d, _NCA, _NCB)
    z = jnp.zeros((rpt, d), jnp.float32)
    return k(y, sidx, ridx, z)


@functools.lru_cache(maxsize=None)
def _make_sc_deg(n, d, n_chunks):
    mesh = plsc.VectorSubcoreMesh(core_axis_name="c", subcore_axis_name="s")
    ncores, nsub = mesh.num_cores, mesh.num_subcores
    npad = _npad(n)
    rpt = npad // nsub
    assert rpt % 8 == 0

    assert n_chunks % (2 * nsub) == 0
    u = n_chunks // (2 * nsub)

    def body(ridx_hbm, z_hbm, ones_hbm, deg_hbm, acc, ridx_v, ones_buf):
        c = lax.axis_index("c")
        s = lax.axis_index("s")
        w = c * nsub + s
        base = s * rpt

        pltpu.sync_copy(z_hbm, acc.at[pl.ds(base, rpt)])
        pltpu.sync_copy(ones_hbm, ones_buf)
        plsc.subcore_barrier()

        def chunk(j, carry):
            pltpu.sync_copy(ridx_hbm.at[w * u + j], ridx_v)
            pltpu.sync_copy(ones_buf, acc.at[ridx_v], add=True)
            return carry

        lax.fori_loop(0, u, chunk, 0)
        plsc.subcore_barrier()
        pltpu.sync_copy(acc.at[pl.ds(base, rpt)],
                        deg_hbm.at[c, pl.ds(base, rpt)])

    return pl.kernel(
        body,
        out_type=jax.ShapeDtypeStruct((ncores, npad, d), jnp.float32),
        mesh=mesh,
        scratch_types=(
            pltpu.VMEM_SHARED((npad, d), jnp.float32),
            pltpu.VMEM((_CH,), jnp.int32),
            pltpu.VMEM((_CH, d), jnp.float32),
        )), npad, rpt


def _sc_degree(ridx, n, d):
    n_chunks = ridx.shape[0]
    k, npad, rpt = _make_sc_deg(n, d, n_chunks)
    z = jnp.zeros((rpt, d), jnp.float32)
    ones_blk = jnp.ones((_CH, d), jnp.float32)
    return k(ridx, z, ones_blk)


def _edge_blocks(senders, receivers, n):
    e = senders.shape[0]
    n_chunks = 16 * (_NCA + _NCB)
    e_pad = n_chunks * _CH
    assert e_pad >= e
    pad = e_pad - e
    sidx = jnp.concatenate(
        [senders, jnp.zeros((pad,), jnp.int32)]).reshape(n_chunks, _CH)
    ridx = jnp.concatenate(
        [receivers, jnp.full((pad,), n, jnp.int32)]).reshape(n_chunks, _CH)
    return sidx, ridx


def kernel(nodes, senders, receivers, n_node, W1, b1, W2, b2, W_out, b_out):
    n, d = nodes.shape
    sidx, ridx = _edge_blocks(senders, receivers, n)

    deg_full = _sc_degree(ridx, n, d)
    degp = deg_full[:, :, :16]
    nodes_b, _ = lax.optimization_barrier((nodes, deg_full))
    agg1p = _sc_aggregate(nodes_b, sidx, ridx)
    x1 = _tc_stage_mid(agg1p, degp, n, b1, W1)
    agg2p = _sc_aggregate(x1, sidx, ridx)
    return _tc_stage_out(agg2p, degp, n, W2, b2, n_node, W_out, b_out)
